# baseline (speedup 1.0000x reference)
import functools

import jax
import jax.numpy as jnp
from jax import lax
from jax.experimental import pallas as pl
from jax.experimental.pallas import tpu as pltpu
from jax.experimental.pallas import tpu_sc as plsc

_B = 16384
_C = 256
_NW = 32
_ROWS_W = _B // _NW
_CHUNK = 128
_NCHUNK = _ROWS_W // _CHUNK
_NGRP = _CHUNK // 16
_SEG = 4
_SEGLEN = _C // _SEG
_XWORDS = _CHUNK * _C
_SEGREF = _XWORDS - _SEGLEN * (_SEG - 1)

_mesh = plsc.VectorSubcoreMesh(core_axis_name="c", subcore_axis_name="s")


@functools.partial(
    pl.kernel,
    out_type=jax.ShapeDtypeStruct((_B * 2,), jnp.float32),
    mesh=_mesh,
    compiler_params=pltpu.CompilerParams(needs_layout_passes=False),
    scratch_types=[
        pltpu.VMEM((_XWORDS,), jnp.float32),
        pltpu.VMEM((_XWORDS,), jnp.float32),
        pltpu.VMEM((_C * 2,), jnp.float32),
        pltpu.VMEM((_ROWS_W * 2,), jnp.float32),
        pltpu.SemaphoreType.DMA,
        pltpu.SemaphoreType.DMA,
    ],
)
def _qam_encode(x_hbm, map_hbm, out_hbm, xbuf0, xbuf1, mapbuf, outbuf,
                sem0, sem1):
    wid = lax.axis_index("s") * 2 + lax.axis_index("c")
    base = wid * _ROWS_W * _C
    sems = (sem0, sem1)
    xbufs = (xbuf0, xbuf1)
    iota = lax.iota(jnp.int32, 16)
    row_off = iota * _C

    copies = [None, None]
    copies[0] = pltpu.async_copy(
        x_hbm.at[pl.ds(base, _XWORDS)], xbufs[0], sems[0])
    pltpu.sync_copy(map_hbm, mapbuf)

    neg_inf = jnp.full((16,), -jnp.inf, jnp.float32)

    for t in range(_NCHUNK):
        nxt = t + 1
        if nxt < _NCHUNK:
            copies[nxt % 2] = pltpu.async_copy(
                x_hbm.at[pl.ds(base + nxt * _XWORDS, _XWORDS)],
                xbufs[nxt % 2], sems[nxt % 2])
        copies[t % 2].wait()
        xb = xbufs[t % 2]

        def grp(g, _, xb=xb, t=t):
            rowflat = g * (16 * _C) + row_off

            def col(_, carry):
                ads = carry[:_SEG]
                ms = carry[_SEG:2 * _SEG]
                bs = carry[2 * _SEG:]
                new_ads, new_ms, new_bs = [], [], []
                for k in range(_SEG):
                    v = plsc.load_gather(xb, [ads[k]])
                    upd = v > ms[k]
                    new_ms.append(jnp.where(upd, v, ms[k]))
                    new_bs.append(jnp.where(upd, ads[k], bs[k]))
                    new_ads.append(ads[k] + 1)
                return (*new_ads, *new_ms, *new_bs)

            a0 = [rowflat + k * _SEGLEN for k in range(_SEG)]
            res = lax.fori_loop(
                0, _SEGLEN, col,
                (*a0, *([neg_inf] * _SEG), *a0),
                unroll=4)
            ms = res[_SEG:2 * _SEG]
            bs = res[2 * _SEG:]

            m_best = ms[0]
            g_best = bs[0] - rowflat
            for k in range(1, _SEG):
                gk = bs[k] - rowflat
                upd = ms[k] > m_best
                m_best = jnp.where(upd, ms[k], m_best)
                g_best = jnp.where(upd, gk, g_best)

            i_val = plsc.load_gather(mapbuf, [g_best * 2])
            q_val = plsc.load_gather(mapbuf, [g_best * 2 + 1])
            pos = (t * _CHUNK + g * 16 + iota) * 2
            plsc.store_scatter(outbuf, [pos], i_val)
            plsc.store_scatter(outbuf, [pos + 1], q_val)
            return 0

        lax.fori_loop(0, _NGRP, grp, 0)

    pltpu.sync_copy(outbuf, out_hbm.at[pl.ds(wid * _ROWS_W * 2, _ROWS_W * 2)])


def kernel(x, mapping):
    out = _qam_encode(x.reshape(-1), mapping.reshape(-1))
    return out.reshape(_B, 2)


# speedup vs baseline: 1.1559x; 1.1559x over previous
import functools

import jax
import jax.numpy as jnp
from jax import lax
from jax.experimental import pallas as pl
from jax.experimental.pallas import tpu as pltpu
from jax.experimental.pallas import tpu_sc as plsc

_B = 16384
_C = 256
_NW = 32
_ROWS_W = _B // _NW
_CHUNK = 128
_NCHUNK = _ROWS_W // _CHUNK
_NGRP = _CHUNK // 16
_SEG = 4
_SEGLEN = _C // _SEG
_PADC = _C + 1

_mesh = plsc.VectorSubcoreMesh(core_axis_name="c", subcore_axis_name="s")


@functools.partial(
    pl.kernel,
    out_type=jax.ShapeDtypeStruct((_B * 2,), jnp.float32),
    mesh=_mesh,
    compiler_params=pltpu.CompilerParams(needs_layout_passes=False),
    scratch_types=[
        pltpu.VMEM((_CHUNK, _PADC), jnp.float32),
        pltpu.VMEM((_CHUNK, _PADC), jnp.float32),
        pltpu.VMEM((_C * 2,), jnp.float32),
        pltpu.VMEM((_ROWS_W * 2,), jnp.float32),
        pltpu.SemaphoreType.DMA,
        pltpu.SemaphoreType.DMA,
    ],
)
def _qam_encode(x_hbm, map_hbm, out_hbm, xbuf0, xbuf1, mapbuf, outbuf,
                sem0, sem1):
    wid = lax.axis_index("s") * 2 + lax.axis_index("c")
    row0 = wid * _ROWS_W
    sems = (sem0, sem1)
    xbufs = (xbuf0, xbuf1)
    iota = lax.iota(jnp.int32, 16)

    copies = [None, None]
    copies[0] = pltpu.async_copy(
        x_hbm.at[pl.ds(row0, _CHUNK), :],
        xbufs[0].at[:, pl.ds(0, _C)], sems[0])
    pltpu.sync_copy(map_hbm, mapbuf)

    neg_inf = jnp.full((16,), -jnp.inf, jnp.float32)

    for t in range(_NCHUNK):
        nxt = t + 1
        if nxt < _NCHUNK:
            copies[nxt % 2] = pltpu.async_copy(
                x_hbm.at[pl.ds(row0 + nxt * _CHUNK, _CHUNK), :],
                xbufs[nxt % 2].at[:, pl.ds(0, _C)], sems[nxt % 2])
        copies[t % 2].wait()
        xb = xbufs[t % 2]

        def grp(g, _, xb=xb, t=t):
            rows = g * 16 + iota

            def col(_, carry):
                cs = carry[:_SEG]
                ms = carry[_SEG:2 * _SEG]
                bs = carry[2 * _SEG:]
                new_cs, new_ms, new_bs = [], [], []
                for k in range(_SEG):
                    v = plsc.load_gather(xb, [rows, cs[k]])
                    upd = v > ms[k]
                    new_ms.append(jnp.where(upd, v, ms[k]))
                    new_bs.append(jnp.where(upd, cs[k], bs[k]))
                    new_cs.append(cs[k] + 1)
                return (*new_cs, *new_ms, *new_bs)

            c0 = [jnp.full((16,), k * _SEGLEN, jnp.int32)
                  for k in range(_SEG)]
            res = lax.fori_loop(
                0, _SEGLEN, col,
                (*c0, *([neg_inf] * _SEG), *c0),
                unroll=4)
            ms = res[_SEG:2 * _SEG]
            bs = res[2 * _SEG:]

            m_best = ms[0]
            g_best = bs[0]
            for k in range(1, _SEG):
                upd = ms[k] > m_best
                m_best = jnp.where(upd, ms[k], m_best)
                g_best = jnp.where(upd, bs[k], g_best)

            i_val = plsc.load_gather(mapbuf, [g_best * 2])
            q_val = plsc.load_gather(mapbuf, [g_best * 2 + 1])
            pos = (t * _CHUNK + g * 16 + iota) * 2
            plsc.store_scatter(outbuf, [pos], i_val)
            plsc.store_scatter(outbuf, [pos + 1], q_val)
            return 0

        lax.fori_loop(0, _NGRP, grp, 0)

    pltpu.sync_copy(outbuf, out_hbm.at[pl.ds(wid * _ROWS_W * 2, _ROWS_W * 2)])


def kernel(x, mapping):
    out = _qam_encode(x, mapping.reshape(-1))
    return out.reshape(_B, 2)


# speedup vs baseline: 1.1576x; 1.0015x over previous
import functools

import jax
import jax.numpy as jnp
from jax import lax
from jax.experimental import pallas as pl
from jax.experimental.pallas import tpu as pltpu
from jax.experimental.pallas import tpu_sc as plsc

_B = 16384
_C = 256
_NW = 32
_ROWS_W = _B // _NW
_CHUNK = 128
_NCHUNK = _ROWS_W // _CHUNK
_NGRP = _CHUNK // 16
_SEG = 4
_SEGLEN = _C // _SEG
_PADC = _C + 8

_mesh = plsc.VectorSubcoreMesh(core_axis_name="c", subcore_axis_name="s")


@functools.partial(
    pl.kernel,
    out_type=jax.ShapeDtypeStruct((_B * 2,), jnp.float32),
    mesh=_mesh,
    compiler_params=pltpu.CompilerParams(needs_layout_passes=False),
    scratch_types=[
        pltpu.VMEM((_CHUNK, _PADC), jnp.float32),
        pltpu.VMEM((_CHUNK, _PADC), jnp.float32),
        pltpu.VMEM((_C * 2,), jnp.float32),
        pltpu.VMEM((_ROWS_W * 2,), jnp.float32),
        pltpu.SemaphoreType.DMA,
        pltpu.SemaphoreType.DMA,
    ],
)
def _qam_encode(x_hbm, map_hbm, out_hbm, xbuf0, xbuf1, mapbuf, outbuf,
                sem0, sem1):
    wid = lax.axis_index("s") * 2 + lax.axis_index("c")
    row0 = wid * _ROWS_W
    sems = (sem0, sem1)
    xbufs = (xbuf0, xbuf1)
    iota = lax.iota(jnp.int32, 16)

    copies = [None, None]
    copies[0] = pltpu.async_copy(
        x_hbm.at[pl.ds(row0, _CHUNK), :],
        xbufs[0].at[:, pl.ds(0, _C)], sems[0])
    pltpu.sync_copy(map_hbm, mapbuf)

    neg_inf = jnp.full((16,), -jnp.inf, jnp.float32)

    for t in range(_NCHUNK):
        nxt = t + 1
        if nxt < _NCHUNK:
            copies[nxt % 2] = pltpu.async_copy(
                x_hbm.at[pl.ds(row0 + nxt * _CHUNK, _CHUNK), :],
                xbufs[nxt % 2].at[:, pl.ds(0, _C)], sems[nxt % 2])
        copies[t % 2].wait()
        xb = xbufs[t % 2]

        def grp(g, _, xb=xb, t=t):
            rows = g * 16 + iota

            def col(_, carry):
                cs = carry[:_SEG]
                ms = carry[_SEG:2 * _SEG]
                bs = carry[2 * _SEG:]
                new_cs, new_ms, new_bs = [], [], []
                for k in range(_SEG):
                    v = plsc.load_gather(xb, [rows, cs[k]])
                    upd = v > ms[k]
                    new_ms.append(jnp.where(upd, v, ms[k]))
                    new_bs.append(jnp.where(upd, cs[k], bs[k]))
                    new_cs.append(cs[k] + 1)
                return (*new_cs, *new_ms, *new_bs)

            c0 = [jnp.full((16,), k * _SEGLEN, jnp.int32)
                  for k in range(_SEG)]
            res = lax.fori_loop(
                0, _SEGLEN, col,
                (*c0, *([neg_inf] * _SEG), *c0),
                unroll=4)
            ms = res[_SEG:2 * _SEG]
            bs = res[2 * _SEG:]

            m_best = ms[0]
            g_best = bs[0]
            for k in range(1, _SEG):
                upd = ms[k] > m_best
                m_best = jnp.where(upd, ms[k], m_best)
                g_best = jnp.where(upd, bs[k], g_best)

            i_val = plsc.load_gather(mapbuf, [g_best * 2])
            q_val = plsc.load_gather(mapbuf, [g_best * 2 + 1])
            pos = (t * _CHUNK + g * 16 + iota) * 2
            plsc.store_scatter(outbuf, [pos], i_val)
            plsc.store_scatter(outbuf, [pos + 1], q_val)
            return 0

        lax.fori_loop(0, _NGRP, grp, 0)

    pltpu.sync_copy(outbuf, out_hbm.at[pl.ds(wid * _ROWS_W * 2, _ROWS_W * 2)])


def kernel(x, mapping):
    out = _qam_encode(x, mapping.reshape(-1))
    return out.reshape(_B, 2)


# speedup vs baseline: 1.8833x; 1.6269x over previous
import functools

import jax
import jax.numpy as jnp
from jax import lax
from jax.experimental import pallas as pl
from jax.experimental.pallas import tpu as pltpu
from jax.experimental.pallas import tpu_sc as plsc

_B = 16384
_C = 256
_NW = 32
_ROWS_W = _B // _NW
_CHUNK = 128
_NCHUNK = _ROWS_W // _CHUNK
_NSTEP = _C // 16

_mesh = plsc.VectorSubcoreMesh(core_axis_name="c", subcore_axis_name="s")


@functools.partial(
    pl.kernel,
    out_type=jax.ShapeDtypeStruct((_B * 2,), jnp.float32),
    mesh=_mesh,
    compiler_params=pltpu.CompilerParams(needs_layout_passes=False),
    scratch_types=[
        pltpu.VMEM((_CHUNK, _C), jnp.float32),
        pltpu.VMEM((_CHUNK, _C), jnp.float32),
        pltpu.VMEM((_C * 2 + 16,), jnp.float32),
        pltpu.VMEM((_ROWS_W * 2,), jnp.float32),
        pltpu.SemaphoreType.DMA,
        pltpu.SemaphoreType.DMA,
    ],
)
def _qam_encode(x_hbm, map_hbm, out_hbm, xbuf0, xbuf1, mapbuf, outbuf,
                sem0, sem1):
    wid = lax.axis_index("s") * 2 + lax.axis_index("c")
    row0 = wid * _ROWS_W
    sems = (sem0, sem1)
    xbufs = (xbuf0, xbuf1)
    iota = lax.iota(jnp.int32, 16)

    copies = [None, None]
    copies[0] = pltpu.async_copy(
        x_hbm.at[pl.ds(row0, _CHUNK), :], xbufs[0], sems[0])
    pltpu.sync_copy(map_hbm, mapbuf.at[pl.ds(0, _C * 2)])

    neg_inf = jnp.full((16,), -jnp.inf, jnp.float32)
    zeros = jnp.zeros((16,), jnp.int32)
    out2 = iota < 2

    for t in range(_NCHUNK):
        nxt = t + 1
        if nxt < _NCHUNK:
            copies[nxt % 2] = pltpu.async_copy(
                x_hbm.at[pl.ds(row0 + nxt * _CHUNK, _CHUNK), :],
                xbufs[nxt % 2], sems[nxt % 2])
        copies[t % 2].wait()
        xb = xbufs[t % 2]

        def row(r, _, xb=xb, t=t):
            def step(j, carry):
                m, jb = carry
                v = xb[r, pl.ds(j * 16, 16)]
                upd = v > m
                m = jnp.where(upd, v, m)
                jb = jnp.where(upd, j, jb)
                return m, jb

            m, jb = lax.fori_loop(0, _NSTEP, step, (neg_inf, zeros),
                                  unroll=4)
            best = lax.reduce_max(m, axes=(0,))
            cand = jnp.where(m == best, jb * 16 + iota, _C)
            imin = lax.reduce_min(cand, axes=(0,))
            ivqv = mapbuf[pl.ds(2 * imin, 16)]
            plsc.store_scatter(
                outbuf, [iota + (t * _CHUNK + r) * 2], ivqv, mask=out2)
            return 0

        lax.fori_loop(0, _CHUNK, row, 0, unroll=2)

    pltpu.sync_copy(outbuf, out_hbm.at[pl.ds(wid * _ROWS_W * 2, _ROWS_W * 2)])


def kernel(x, mapping):
    out = _qam_encode(x, mapping.reshape(-1))
    return out.reshape(_B, 2)


# speedup vs baseline: 1.9460x; 1.0333x over previous
import functools

import jax
import jax.numpy as jnp
from jax import lax
from jax.experimental import pallas as pl
from jax.experimental.pallas import tpu as pltpu
from jax.experimental.pallas import tpu_sc as plsc

_B = 16384
_C = 256
_NW = 32
_ROWS_W = _B // _NW
_CHUNK = 128
_NCHUNK = _ROWS_W // _CHUNK
_NSTEP = _C // 16

_mesh = plsc.VectorSubcoreMesh(core_axis_name="c", subcore_axis_name="s")


@functools.partial(
    pl.kernel,
    out_type=jax.ShapeDtypeStruct((_B * 2,), jnp.float32),
    mesh=_mesh,
    compiler_params=pltpu.CompilerParams(needs_layout_passes=False),
    scratch_types=[
        pltpu.VMEM((_CHUNK, _C), jnp.float32),
        pltpu.VMEM((_CHUNK, _C), jnp.float32),
        pltpu.VMEM((_C * 2 + 16,), jnp.float32),
        pltpu.VMEM((_ROWS_W * 2,), jnp.float32),
        pltpu.SemaphoreType.DMA,
        pltpu.SemaphoreType.DMA,
    ],
)
def _qam_encode(x_hbm, map_hbm, out_hbm, xbuf0, xbuf1, mapbuf, outbuf,
                sem0, sem1):
    wid = lax.axis_index("s") * 2 + lax.axis_index("c")
    row0 = wid * _ROWS_W
    sems = (sem0, sem1)
    xbufs = (xbuf0, xbuf1)
    iota = lax.iota(jnp.int32, 16)

    copies = [None, None]
    copies[0] = pltpu.async_copy(
        x_hbm.at[pl.ds(row0, _CHUNK), :], xbufs[0], sems[0])
    pltpu.sync_copy(map_hbm, mapbuf.at[pl.ds(0, _C * 2)])

    neg_inf = jnp.full((16,), -jnp.inf, jnp.float32)
    zeros = jnp.zeros((16,), jnp.int32)
    out2 = iota < 2

    for t in range(_NCHUNK):
        nxt = t + 1
        if nxt < _NCHUNK:
            copies[nxt % 2] = pltpu.async_copy(
                x_hbm.at[pl.ds(row0 + nxt * _CHUNK, _CHUNK), :],
                xbufs[nxt % 2], sems[nxt % 2])
        copies[t % 2].wait()
        xb = xbufs[t % 2]

        def row(r, _, xb=xb, t=t):
            def step(j, carry):
                m0, jb0, m1, jb1 = carry
                v0 = xb[r, pl.ds(j * 16, 16)]
                v1 = xb[r, pl.ds(j * 16 + _C // 2, 16)]
                u0 = v0 > m0
                u1 = v1 > m1
                m0 = jnp.where(u0, v0, m0)
                jb0 = jnp.where(u0, j, jb0)
                m1 = jnp.where(u1, v1, m1)
                jb1 = jnp.where(u1, j, jb1)
                return m0, jb0, m1, jb1

            m0, jb0, m1, jb1 = lax.fori_loop(
                0, _NSTEP // 2, step, (neg_inf, zeros, neg_inf, zeros),
                unroll=4)
            c0 = jb0 * 16 + iota
            c1 = jb1 * 16 + iota + _C // 2
            u = m1 > m0
            m = jnp.where(u, m1, m0)
            c = jnp.where(u, c1, c0)
            best = lax.reduce_max(m, axes=(0,))
            cand = jnp.where(m == best, c, _C)
            imin = lax.reduce_min(cand, axes=(0,))
            ivqv = mapbuf[pl.ds(2 * imin, 16)]
            plsc.store_scatter(
                outbuf, [iota + (t * _CHUNK + r) * 2], ivqv, mask=out2)
            return 0

        lax.fori_loop(0, _CHUNK, row, 0, unroll=4)

    pltpu.sync_copy(outbuf, out_hbm.at[pl.ds(wid * _ROWS_W * 2, _ROWS_W * 2)])


def kernel(x, mapping):
    out = _qam_encode(x, mapping.reshape(-1))
    return out.reshape(_B, 2)
